# Initial kernel scaffold; baseline (speedup 1.0000x reference)
import jax
import jax.numpy as jnp
from jax.experimental import pallas as pl


def kernel(x, adj_logits, W1, b1, gamma1, beta1, W2, b2, gamma2, beta2, W3, b3):
    raise NotImplementedError("write your pallas kernel here")


# speedup vs baseline: 62.5238x; 62.5238x over previous
import functools

import jax
import jax.numpy as jnp
from jax.experimental import pallas as pl
from jax.experimental.pallas import tpu as pltpu

B = 100
N = 200
FLAT = N * N
NUM_EL = int(0.3 * N * N)
H1 = 512
H2 = 1024
OUT = 2
EPS = 1e-5

KBLK = 2048
KT = (FLAT + KBLK - 1) // KBLK


def _thresh_body(adj_ref, thr_ref):
    a = adj_ref[...]
    keys = jax.lax.bitcast_convert_type(jnp.abs(a), jnp.int32)

    def bit_step(i, t):
        cand = t | jnp.left_shift(1, 30 - i)
        cnt = jnp.sum((keys >= cand).astype(jnp.int32), axis=1, keepdims=True)
        return jnp.where(cnt >= NUM_EL, cand, t)

    t = jax.lax.fori_loop(0, 31, bit_step, jnp.zeros((B, 1), jnp.int32))
    thr = jax.lax.bitcast_convert_type(t, jnp.float32)
    thr_ref[...] = jnp.broadcast_to(thr, (B, 128))


def _mlp_body(x_ref, adj_ref, thr_ref, w1_ref, b1_ref, g1_ref, be1_ref,
              w2_ref, b2_ref, g2_ref, be2_ref, w3_ref, b3_ref,
              out_ref, acc_ref):
    i = pl.program_id(0)

    @pl.when(i == 0)
    def _():
        acc_ref[...] = jnp.zeros_like(acc_ref)

    thr = thr_ref[:, 0:1]
    col = i * KBLK + jax.lax.broadcasted_iota(jnp.int32, (B, KBLK), 1)
    keep = (col < FLAT) & (jnp.abs(adj_ref[...]) >= thr)
    xm = jnp.where(keep, x_ref[...], 0.0)
    row = i * KBLK + jax.lax.broadcasted_iota(jnp.int32, (KBLK, H1), 0)
    w1 = jnp.where(row < FLAT, w1_ref[...], 0.0)
    acc_ref[...] += jnp.dot(xm, w1, preferred_element_type=jnp.float32)

    @pl.when(i == KT - 1)
    def _():
        s = 1.0 / (1.0 + EPS) ** 0.5
        h = jnp.maximum(acc_ref[...] + b1_ref[...], 0.0)
        h = g1_ref[...] * h * s + be1_ref[...]
        h = jnp.maximum(jnp.dot(h, w2_ref[...], preferred_element_type=jnp.float32)
                        + b2_ref[...], 0.0)
        h = g2_ref[...] * h * s + be2_ref[...]
        lg = jnp.dot(h, w3_ref[...], preferred_element_type=jnp.float32) + b3_ref[...]
        c = jax.lax.broadcasted_iota(jnp.int32, lg.shape, 1)
        neg = jnp.where(c < OUT, lg, -jnp.inf)
        m = jnp.max(neg, axis=1, keepdims=True)
        ex = jnp.where(c < OUT, jnp.exp(lg - m), 0.0)
        lse = m + jnp.log(jnp.sum(ex, axis=1, keepdims=True))
        out_ref[...] = lg - lse


def kernel(x, adj_logits, W1, b1, gamma1, beta1, W2, b2, gamma2, beta2, W3, b3):
    adj = adj_logits.reshape(B, FLAT)

    thr = pl.pallas_call(
        _thresh_body,
        out_shape=jax.ShapeDtypeStruct((B, 128), jnp.float32),
    )(adj)

    w3p = jnp.pad(W3, ((0, 0), (0, 128 - OUT)))
    b3p = jnp.pad(b3, (0, 128 - OUT)).reshape(1, 128)

    out = pl.pallas_call(
        _mlp_body,
        grid=(KT,),
        in_specs=[
            pl.BlockSpec((B, KBLK), lambda i: (0, i)),
            pl.BlockSpec((B, KBLK), lambda i: (0, i)),
            pl.BlockSpec((B, 128), lambda i: (0, 0)),
            pl.BlockSpec((KBLK, H1), lambda i: (i, 0)),
            pl.BlockSpec((1, H1), lambda i: (0, 0)),
            pl.BlockSpec((1, H1), lambda i: (0, 0)),
            pl.BlockSpec((1, H1), lambda i: (0, 0)),
            pl.BlockSpec((H1, H2), lambda i: (0, 0)),
            pl.BlockSpec((1, H2), lambda i: (0, 0)),
            pl.BlockSpec((1, H2), lambda i: (0, 0)),
            pl.BlockSpec((1, H2), lambda i: (0, 0)),
            pl.BlockSpec((H2, 128), lambda i: (0, 0)),
            pl.BlockSpec((1, 128), lambda i: (0, 0)),
        ],
        out_specs=pl.BlockSpec((B, 128), lambda i: (0, 0)),
        out_shape=jax.ShapeDtypeStruct((B, 128), jnp.float32),
        scratch_shapes=[pltpu.VMEM((B, H1), jnp.float32)],
    )(x, adj, thr, W1, b1.reshape(1, H1), gamma1.reshape(1, H1),
      beta1.reshape(1, H1), W2, b2.reshape(1, H2), gamma2.reshape(1, H2),
      beta2.reshape(1, H2), w3p, b3p)

    return out[:, :OUT]
